# baseline (speedup 1.0000x reference)
import functools

import jax
import jax.numpy as jnp
from jax import lax
from jax.experimental import pallas as pl
from jax.experimental.pallas import tpu as pltpu
from jax.experimental.pallas import tpu_sc as plsc

_B, _T = 16384, 200
_V, _D = 100, 16
_DP = _D + 1
_NW = 32
_W = _B // _NW
_TB = 8
_NTB = _T // _TB
_G = _W // 16

_mesh = plsc.VectorSubcoreMesh(core_axis_name="c", subcore_axis_name="s")


@functools.partial(
    pl.kernel,
    mesh=_mesh,
    out_type=jax.ShapeDtypeStruct((_T, _D, _B), jnp.float32),
    scratch_types=[
        pltpu.VMEM((_V * _DP,), jnp.float32),
        pltpu.VMEM((_TB, _W), jnp.int32),
        pltpu.VMEM((_D, _W), jnp.float32),
        pltpu.VMEM((_D, _W), jnp.float32),
        pltpu.SemaphoreType.DMA,
        pltpu.SemaphoreType.DMA,
        pltpu.SemaphoreType.DMA,
    ],
    compiler_params=pltpu.CompilerParams(
        use_tc_tiling_on_sc=True, needs_layout_passes=False
    ),
)
def _emb_lookup(idx_hbm, table_hbm, out_hbm, table_v, idx_v, slab0, slab1,
                sem_t, sem0, sem1):
    wid = lax.axis_index("s") * 2 + lax.axis_index("c")
    b0 = wid * _W
    pltpu.async_copy(table_hbm, table_v, sem_t).wait()
    slabs = (slab0, slab1)
    sems = (sem0, sem1)

    @pl.loop(0, _NTB)
    def _tblock(tb):
        t0 = tb * _TB
        pltpu.sync_copy(idx_hbm.at[pl.ds(t0, _TB), pl.ds(b0, _W)], idx_v)
        for tl in range(_TB):
            slab = slabs[tl % 2]
            sem = sems[tl % 2]
            dst = out_hbm.at[t0 + tl, :, pl.ds(b0, _W)]

            @pl.when(jnp.logical_or(tb > 0, tl >= 2))
            def _drain():
                pltpu.make_async_copy(slab, dst, sem).wait()

            @plsc.parallel_loop(0, _G, unroll=8)
            def _group(g):
                iv = idx_v[tl, pl.ds(g * 16, 16)]
                base = iv * _DP
                for d in range(_D):
                    vals = plsc.load_gather(table_v, [base + d])
                    slab[d, pl.ds(g * 16, 16)] = vals

            pltpu.async_copy(slab, dst, sem)

    last = out_hbm.at[_T - 1, :, pl.ds(b0, _W)]
    pltpu.make_async_copy(slab0, last, sem0).wait()
    pltpu.make_async_copy(slab1, last, sem1).wait()


def kernel(src, emb_table):
    idx_t = jnp.swapaxes(src, 0, 1).astype(jnp.int32)
    table_p = jnp.pad(emb_table, ((0, 0), (0, 1))).reshape(-1)
    out = _emb_lookup(idx_t, table_p)
    return jnp.transpose(out, (2, 0, 1))


# speedup vs baseline: 1.2907x; 1.2907x over previous
import functools

import jax
import jax.numpy as jnp
from jax import lax
from jax.experimental import pallas as pl
from jax.experimental.pallas import tpu as pltpu
from jax.experimental.pallas import tpu_sc as plsc

_B, _T = 16384, 200
_V, _D = 100, 16
_DP = _D + 1
_NW = 32
_W = _B // _NW
_TB = 8
_NTB = _T // _TB
_G = _W // 16

_mesh = plsc.VectorSubcoreMesh(core_axis_name="c", subcore_axis_name="s")


@functools.partial(
    pl.kernel,
    mesh=_mesh,
    out_type=jax.ShapeDtypeStruct((_T, _D, _B), jnp.float32),
    scratch_types=[
        pltpu.VMEM((_V * _DP,), jnp.float32),
        pltpu.VMEM((_TB, _W), jnp.int32),
        pltpu.VMEM((_D, _W), jnp.float32),
        pltpu.VMEM((_D, _W), jnp.float32),
        pltpu.SemaphoreType.DMA,
        pltpu.SemaphoreType.DMA,
        pltpu.SemaphoreType.DMA,
    ],
    compiler_params=pltpu.CompilerParams(
        use_tc_tiling_on_sc=True, needs_layout_passes=False
    ),
)
def _emb_lookup(idx_hbm, table_hbm, out_hbm, table_v, idx_v, slab0, slab1,
                sem_t, sem0, sem1):
    wid = lax.axis_index("s") * 2 + lax.axis_index("c")
    b0 = wid * _W
    pltpu.async_copy(table_hbm, table_v, sem_t).wait()
    slabs = (slab0, slab1)
    sems = (sem0, sem1)

    @pl.loop(0, _NTB)
    def _tblock(tb):
        t0 = tb * _TB
        pltpu.sync_copy(idx_hbm.at[pl.ds(t0, _TB), pl.ds(b0, _W)], idx_v)
        for tl in range(_TB):
            slab = slabs[tl % 2]
            sem = sems[tl % 2]
            dst = out_hbm.at[t0 + tl, :, pl.ds(b0, _W)]

            @pl.when(jnp.logical_or(tb > 0, tl >= 2))
            def _drain():
                pltpu.make_async_copy(slab, dst, sem).wait()

            @plsc.parallel_loop(0, _G, unroll=2)
            def _group(g):
                iv = idx_v[tl, pl.ds(g * 16, 16)]
                base = iv * _DP
                for d in range(_D):
                    vals = plsc.load_gather(table_v, [base + d])
                    slab[d, pl.ds(g * 16, 16)] = vals

            pltpu.async_copy(slab, dst, sem)

    last = out_hbm.at[_T - 1, :, pl.ds(b0, _W)]
    pltpu.make_async_copy(slab0, last, sem0).wait()
    pltpu.make_async_copy(slab1, last, sem1).wait()


def kernel(src, emb_table):
    idx_t = jnp.swapaxes(src, 0, 1).astype(jnp.int32)
    table_p = jnp.pad(emb_table, ((0, 0), (0, 1))).reshape(-1)
    out = _emb_lookup(idx_t, table_p)
    return jnp.transpose(out, (2, 0, 1))


# speedup vs baseline: 1.4443x; 1.1190x over previous
import functools

import jax
import jax.numpy as jnp
from jax import lax
from jax.experimental import pallas as pl
from jax.experimental.pallas import tpu as pltpu
from jax.experimental.pallas import tpu_sc as plsc

_B, _T = 16384, 200
_V, _D = 100, 16
_DP = _D + 1
_NW = 32
_W = _B // _NW
_TB = 8
_NTB = _T // _TB
_G = _W // 16

_mesh = plsc.VectorSubcoreMesh(core_axis_name="c", subcore_axis_name="s")


@functools.partial(
    pl.kernel,
    mesh=_mesh,
    out_type=jax.ShapeDtypeStruct((_T, _D, _B), jnp.float32),
    scratch_types=[
        pltpu.VMEM((_V * _DP,), jnp.float32),
        pltpu.VMEM((_TB, _W), jnp.int32),
        pltpu.VMEM((_D, _W), jnp.float32),
        pltpu.VMEM((_D, _W), jnp.float32),
        pltpu.SemaphoreType.DMA,
        pltpu.SemaphoreType.DMA,
        pltpu.SemaphoreType.DMA,
    ],
    compiler_params=pltpu.CompilerParams(
        use_tc_tiling_on_sc=True, needs_layout_passes=False
    ),
)
def _emb_lookup(idx_hbm, table_hbm, out_hbm, table_v, idx_v, slab0, slab1,
                sem_t, sem0, sem1):
    wid = lax.axis_index("s") * 2 + lax.axis_index("c")
    b0 = wid * _W
    pltpu.async_copy(table_hbm, table_v, sem_t).wait()
    slabs = (slab0, slab1)
    sems = (sem0, sem1)

    @pl.loop(0, _NTB)
    def _tblock(tb):
        t0 = tb * _TB
        pltpu.sync_copy(idx_hbm.at[pl.ds(t0, _TB), pl.ds(b0, _W)], idx_v)
        for tl in range(_TB):
            slab = slabs[tl % 2]
            sem = sems[tl % 2]
            dst = out_hbm.at[t0 + tl, :, pl.ds(b0, _W)]

            @pl.when(jnp.logical_or(tb > 0, tl >= 2))
            def _drain():
                pltpu.make_async_copy(slab, dst, sem).wait()

            @plsc.parallel_loop(0, _G, unroll=1)
            def _group(g):
                iv = idx_v[tl, pl.ds(g * 16, 16)]
                base = iv * _DP
                for d in range(_D):
                    vals = plsc.load_gather(table_v, [base + d])
                    slab[d, pl.ds(g * 16, 16)] = vals

            pltpu.async_copy(slab, dst, sem)

    last = out_hbm.at[_T - 1, :, pl.ds(b0, _W)]
    pltpu.make_async_copy(slab0, last, sem0).wait()
    pltpu.make_async_copy(slab1, last, sem1).wait()


def kernel(src, emb_table):
    idx_t = jnp.swapaxes(src, 0, 1).astype(jnp.int32)
    table_p = jnp.pad(emb_table, ((0, 0), (0, 1))).reshape(-1)
    out = _emb_lookup(idx_t, table_p)
    return jnp.transpose(out, (2, 0, 1))


# speedup vs baseline: 1.5953x; 1.1045x over previous
import functools

import jax
import jax.numpy as jnp
from jax import lax
from jax.experimental import pallas as pl
from jax.experimental.pallas import tpu as pltpu
from jax.experimental.pallas import tpu_sc as plsc

_B, _T = 16384, 200
_V, _D = 100, 16
_DP = _D + 1
_NW = 32
_W = _B // _NW
_TB = 8
_NTB = _T // _TB
_G = _W // 16

_mesh = plsc.VectorSubcoreMesh(core_axis_name="c", subcore_axis_name="s")


@functools.partial(
    pl.kernel,
    mesh=_mesh,
    out_type=jax.ShapeDtypeStruct((_T, _D, _B), jnp.float32),
    scratch_types=[
        pltpu.VMEM((_V * _DP,), jnp.float32),
        pltpu.VMEM((_T, _W), jnp.int32),
        pltpu.VMEM((_D, _W), jnp.float32),
        pltpu.VMEM((_D, _W), jnp.float32),
        pltpu.SemaphoreType.DMA,
        pltpu.SemaphoreType.DMA,
        pltpu.SemaphoreType.DMA,
    ],
    compiler_params=pltpu.CompilerParams(
        use_tc_tiling_on_sc=True, needs_layout_passes=False
    ),
)
def _emb_lookup(idx_hbm, table_hbm, out_hbm, table_v, idx_v, slab0, slab1,
                sem_t, sem0, sem1):
    wid = lax.axis_index("s") * 2 + lax.axis_index("c")
    b0 = wid * _W
    pltpu.async_copy(table_hbm, table_v, sem_t).wait()
    slabs = (slab0, slab1)
    sems = (sem0, sem1)

    pltpu.sync_copy(idx_hbm.at[:, pl.ds(b0, _W)], idx_v)

    @pl.loop(0, _NTB)
    def _tblock(tb):
        t0 = tb * _TB
        for tl in range(_TB):
            slab = slabs[tl % 2]
            sem = sems[tl % 2]
            dst = out_hbm.at[t0 + tl, :, pl.ds(b0, _W)]

            @pl.when(jnp.logical_or(tb > 0, tl >= 2))
            def _drain():
                pltpu.make_async_copy(slab, dst, sem).wait()

            @plsc.parallel_loop(0, _G, unroll=1)
            def _group(g):
                iv = idx_v[t0 + tl, pl.ds(g * 16, 16)]
                base = iv * _DP
                for d in range(_D):
                    vals = plsc.load_gather(table_v, [base + d])
                    slab[d, pl.ds(g * 16, 16)] = vals

            pltpu.async_copy(slab, dst, sem)

    last = out_hbm.at[_T - 1, :, pl.ds(b0, _W)]
    pltpu.make_async_copy(slab0, last, sem0).wait()
    pltpu.make_async_copy(slab1, last, sem1).wait()


def kernel(src, emb_table):
    idx_t = jnp.swapaxes(src, 0, 1).astype(jnp.int32)
    table_p = jnp.pad(emb_table, ((0, 0), (0, 1))).reshape(-1)
    out = _emb_lookup(idx_t, table_p)
    return jnp.transpose(out, (2, 0, 1))
